# baseline (speedup 1.0000x reference)
import jax
import jax.numpy as jnp
from jax.experimental import pallas as pl
from jax.experimental.pallas import tpu as pltpu

_B, _T, _D_AUDIO = 16, 1500, 128
_D_MODEL = 1024
_N_START, _N_END = 4, 1
_T_OUT = _N_START + _T + _N_END
_T_PAD = _N_START + _T


def _copies(y_ref, out_ref, sems, slot, b):
    big = pltpu.make_async_copy(
        y_ref.at[slot, pl.ds(0, _T_PAD), :],
        out_ref.at[b, pl.ds(0, _T_PAD), :],
        sems.at[slot, 0],
    )
    last = pltpu.make_async_copy(
        y_ref.at[slot, pl.ds(_T_PAD, 1), :],
        out_ref.at[b, pl.ds(_T_PAD, 1), :],
        sems.at[slot, 1],
    )
    return big, last


def _body(start_ids_ref, end_ids_ref, sounds_hbm, masks_ref, w_ref,
          embed_ref, out_ref, x_ref, y_ref, emb_ref, x_all_ref, sems):
    b = pl.program_id(0)
    slot = jax.lax.rem(b, 2)

    @pl.when(b == 0)
    def _init():
        in_cp = pltpu.make_async_copy(sounds_hbm, x_all_ref, sems.at[0, 3])
        in_cp.start()
        copies = []
        for j in range(_N_START):
            cp = pltpu.make_async_copy(
                embed_ref.at[pl.ds(start_ids_ref[j], 1), :],
                emb_ref.at[pl.ds(j, 1), :],
                sems.at[0, 2],
            )
            cp.start()
            copies.append(cp)
        for j in range(_N_END):
            cp = pltpu.make_async_copy(
                embed_ref.at[pl.ds(end_ids_ref[j], 1), :],
                emb_ref.at[pl.ds(_N_START + j, 1), :],
                sems.at[1, 2],
            )
            cp.start()
            copies.append(cp)
        x_ref[0:_N_START, :] = jnp.zeros((_N_START, _D_AUDIO), jnp.float32)
        for cp in copies:
            cp.wait()
        in_cp.wait()

    @pl.when(b >= 2)
    def _wait_prev():
        big, last = _copies(y_ref, out_ref, sems, slot, b)
        big.wait()
        last.wait()

    x_ref[_N_START:_T_PAD, :] = x_all_ref[b] * masks_ref[0, 0][:, None]
    y_ref[slot, 0:_T_PAD, :] = jnp.dot(
        x_ref[...], w_ref[...], preferred_element_type=jnp.float32)
    y_ref[slot, 0:_N_START, :] = emb_ref[0:_N_START, :]
    y_ref[slot, _T_PAD:_T_PAD + 1, :] = emb_ref[_N_START:_N_START + 1, :]

    big, last = _copies(y_ref, out_ref, sems, slot, b)
    big.start()
    last.start()

    @pl.when(b == _B - 1)
    def _drain():
        for s in (1 - slot, slot):
            wbig, wlast = _copies(y_ref, out_ref, sems, s, b)
            wbig.wait()
            wlast.wait()


def kernel(sounds, masks, start_token_ids, end_token_ids, embed_table, W_enc):
    masks3 = masks.reshape(_B, 1, _T)
    grid_spec = pltpu.PrefetchScalarGridSpec(
        num_scalar_prefetch=2,
        grid=(_B,),
        in_specs=[
            pl.BlockSpec(memory_space=pltpu.MemorySpace.HBM),
            pl.BlockSpec((1, 1, _T), lambda b, *_: (b, 0, 0)),
            pl.BlockSpec((_D_AUDIO, _D_MODEL), lambda b, *_: (0, 0)),
            pl.BlockSpec(memory_space=pltpu.MemorySpace.HBM),
        ],
        out_specs=pl.BlockSpec(memory_space=pltpu.MemorySpace.HBM),
        scratch_shapes=[
            pltpu.VMEM((_T_PAD, _D_AUDIO), jnp.float32),
            pltpu.VMEM((2, _T_PAD + 8, _D_MODEL), jnp.float32),
            pltpu.VMEM((8, _D_MODEL), jnp.float32),
            pltpu.VMEM((_B, _T, _D_AUDIO), jnp.float32),
            pltpu.SemaphoreType.DMA((2, 4)),
        ],
    )
    return pl.pallas_call(
        _body,
        grid_spec=grid_spec,
        out_shape=jax.ShapeDtypeStruct((_B, _T_OUT, _D_MODEL), jnp.float32),
        compiler_params=pltpu.CompilerParams(
            dimension_semantics=("arbitrary",)),
    )(start_token_ids.astype(jnp.int32), end_token_ids.astype(jnp.int32),
      sounds, masks3, W_enc, embed_table)
